# Initial kernel scaffold; baseline (speedup 1.0000x reference)
import jax
import jax.numpy as jnp
from jax.experimental import pallas as pl


def kernel(encoded_patches, pos_table):
    raise NotImplementedError("write your pallas kernel here")


# speedup vs baseline: 1.0115x; 1.0115x over previous
import jax
import jax.numpy as jnp
from jax.experimental import pallas as pl

NP_ = 576
PD_ = 768
B_ = 256
BLOCK_B = 8


def _add_kernel(x_ref, t_ref, o_ref):
    o_ref[...] = x_ref[...] + t_ref[...]


def kernel(encoded_patches, pos_table):
    grid = (B_ // BLOCK_B,)
    return pl.pallas_call(
        _add_kernel,
        grid=grid,
        in_specs=[
            pl.BlockSpec((BLOCK_B, NP_, PD_), lambda i: (i, 0, 0)),
            pl.BlockSpec((NP_, PD_), lambda i: (0, 0)),
        ],
        out_specs=pl.BlockSpec((BLOCK_B, NP_, PD_), lambda i: (i, 0, 0)),
        out_shape=jax.ShapeDtypeStruct((B_, NP_, PD_), jnp.float32),
    )(encoded_patches, pos_table)
